# baseline (speedup 1.0000x reference)
import functools

import jax
import jax.numpy as jnp
from jax import lax
from jax.experimental import pallas as pl
from jax.experimental.pallas import tpu as pltpu
from jax.experimental.pallas import tpu_sc as plsc

NUM_INPUTS = 4096
ADDRESS_SIZE = 4
NUM_LUTS = NUM_INPUTS // ADDRESS_SIZE
ENTRY_PER_LUT = 2 ** ADDRESS_SIZE
BATCH = 4096

_INFO = plsc.get_sparse_core_info()
_NC = _INFO.num_cores
_NS = _INFO.num_subcores
_L = _INFO.num_lanes
_NW = _NC * _NS
_ROWS_PER_W = BATCH // _NW
_G = 16
_NCHUNKS = _ROWS_PER_W // _G
_NH = 2
_W = NUM_INPUTS // _NH
_LH = NUM_LUTS // _NH
_J = _LH // _L


def _make_kernel():
    mesh = plsc.VectorSubcoreMesh(core_axis_name="c", subcore_axis_name="s")

    @functools.partial(
        pl.kernel,
        mesh=mesh,
        compiler_params=pltpu.CompilerParams(needs_layout_passes=False),
        out_type=jax.ShapeDtypeStruct((BATCH,), jnp.float32),
        scratch_types=[
            pltpu.VMEM((ENTRY_PER_LUT * NUM_LUTS,), jnp.float32),
            pltpu.VMEM((_G, _W), jnp.int32),
            pltpu.VMEM((_G, _W), jnp.int32),
            pltpu.VMEM((_G, _W), jnp.int32),
            pltpu.VMEM((_G,), jnp.float32),
            pltpu.SemaphoreType.DMA,
            pltpu.SemaphoreType.DMA,
            pltpu.SemaphoreType.DMA,
            pltpu.SemaphoreType.DMA,
        ],
    )
    def k(x_hbm, lutst_hbm, out_hbm, luts_v, xa_v, xb_v, xc_v, out_v,
          sa, sb, sc, sl):
        wid = lax.axis_index("s") * _NC + lax.axis_index("c")
        row_base = wid * _ROWS_PER_W
        luts_cpy = pltpu.async_copy(lutst_hbm, luts_v, sl)
        lane = lax.broadcasted_iota(jnp.int32, (_L,), 0)
        shiftv = [((lane >> 2) + i) & 3 for i in range(ADDRESS_SIZE)]
        patv = [lane * ADDRESS_SIZE + shiftv[i] for i in range(ADDRESS_SIZE)]

        bufs = (xa_v, xb_v, xc_v)
        sems = (sa, sb, sc)
        _NB = len(bufs)
        tasks = [(g, h) for g in range(_NCHUNKS) for h in range(_NH)]

        def start(t):
            g, h = tasks[t]
            return pltpu.async_copy(
                x_hbm.at[pl.ds(row_base + g * _G, _G), pl.ds(h * _W, _W)],
                bufs[t % _NB], sems[t % _NB])

        copies = [start(0), start(1)]
        luts_cpy.wait()
        for t, (g, h) in enumerate(tasks):
            xg_v = bufs[t % _NB]
            copies[t].wait()
            if t + 2 < len(tasks):
                copies.append(start(t + 2))
            lbase = h * _LH

            if h == 0:
                out_v[...] = jnp.zeros((_L,), jnp.float32)

            @plsc.parallel_loop(0, _G, step=1, unroll=1)
            def _rows(r):
                rvec = jnp.broadcast_to(r, (_L,))

                @plsc.parallel_loop(0, _J, step=1, unroll=4,
                                    carry=jnp.zeros((_L,), jnp.float32))
                def acc(j, a):
                    cb = jnp.broadcast_to(j * (_L * ADDRESS_SIZE), (_L,))
                    addr = jnp.zeros((_L,), jnp.int32)
                    for i in range(ADDRESS_SIZE):
                        g_i = plsc.load_gather(xg_v, [rvec, cb + patv[i]])
                        addr = addr + (g_i << shiftv[i])
                    lidx = (addr << 10) + jnp.broadcast_to(
                        lbase + j * _L, (_L,)) + lane
                    return a + plsc.load_gather(luts_v, [lidx])

                s = jnp.sum(acc)
                cur = plsc.load_gather(out_v, [rvec])
                plsc.store_scatter(out_v, [rvec], cur + s,
                                   mask=(lane == 0))
            if h == _NH - 1:
                pltpu.sync_copy(
                    out_v, out_hbm.at[pl.ds(row_base + g * _G, _G)])

    return k


_kernel_call = _make_kernel()


@jax.jit
def kernel(x, luts):
    luts_t = luts.reshape(NUM_LUTS, ENTRY_PER_LUT).T.reshape(-1)
    out = _kernel_call(x, luts_t)
    return out.reshape(BATCH, 1)


# speedup vs baseline: 1.0255x; 1.0255x over previous
import functools

import jax
import jax.numpy as jnp
from jax import lax
from jax.experimental import pallas as pl
from jax.experimental.pallas import tpu as pltpu
from jax.experimental.pallas import tpu_sc as plsc

NUM_INPUTS = 4096
ADDRESS_SIZE = 4
NUM_LUTS = NUM_INPUTS // ADDRESS_SIZE
ENTRY_PER_LUT = 2 ** ADDRESS_SIZE
BATCH = 4096

_INFO = plsc.get_sparse_core_info()
_NC = _INFO.num_cores
_NS = _INFO.num_subcores
_L = _INFO.num_lanes
_NW = _NC * _NS
_ROWS_PER_W = BATCH // _NW
_G = 16
_NCHUNKS = _ROWS_PER_W // _G
_NH = 2
_W = NUM_INPUTS // _NH
_LH = NUM_LUTS // _NH
_J = _LH // _L


def _make_kernel():
    mesh = plsc.VectorSubcoreMesh(core_axis_name="c", subcore_axis_name="s")

    @functools.partial(
        pl.kernel,
        mesh=mesh,
        compiler_params=pltpu.CompilerParams(needs_layout_passes=False),
        out_type=jax.ShapeDtypeStruct((BATCH,), jnp.float32),
        scratch_types=[
            pltpu.VMEM((ENTRY_PER_LUT * NUM_LUTS,), jnp.float32),
            pltpu.VMEM((_G, _W), jnp.int32),
            pltpu.VMEM((_G, _W), jnp.int32),
            pltpu.VMEM((_G,), jnp.float32),
            pltpu.SemaphoreType.DMA,
            pltpu.SemaphoreType.DMA,
            pltpu.SemaphoreType.DMA,
        ],
    )
    def k(x_hbm, lutst_hbm, out_hbm, luts_v, xa_v, xb_v, out_v, sa, sb, sl):
        wid = lax.axis_index("s") * _NC + lax.axis_index("c")
        row_base = wid * _ROWS_PER_W
        luts_cpy = pltpu.async_copy(lutst_hbm, luts_v, sl)
        lane = lax.broadcasted_iota(jnp.int32, (_L,), 0)
        shiftv = [((lane >> 2) + i) & 3 for i in range(ADDRESS_SIZE)]
        patv = [lane * ADDRESS_SIZE + shiftv[i] for i in range(ADDRESS_SIZE)]

        bufs = (xa_v, xb_v)
        sems = (sa, sb)
        tasks = [(g, h) for g in range(_NCHUNKS) for h in range(_NH)]

        def start(t):
            g, h = tasks[t]
            return pltpu.async_copy(
                x_hbm.at[pl.ds(row_base + g * _G, _G), pl.ds(h * _W, _W)],
                bufs[t % 2], sems[t % 2])

        pending = start(0)
        luts_cpy.wait()
        for t, (g, h) in enumerate(tasks):
            xg_v = bufs[t % 2]
            pending.wait()
            if t + 1 < len(tasks):
                pending = start(t + 1)
            lbase = h * _LH

            if h == 0:
                out_v[...] = jnp.zeros((_L,), jnp.float32)

            @plsc.parallel_loop(0, _G, step=1, unroll=1)
            def _rows(r):
                rvec = jnp.broadcast_to(r, (_L,))

                @plsc.parallel_loop(0, _J, step=1, unroll=4,
                                    carry=jnp.zeros((_L,), jnp.float32))
                def acc(j, a):
                    cb = jnp.broadcast_to(j * (_L * ADDRESS_SIZE), (_L,))
                    addr = jnp.zeros((_L,), jnp.int32)
                    for i in range(ADDRESS_SIZE):
                        g_i = plsc.load_gather(xg_v, [rvec, cb + patv[i]])
                        addr = addr + (g_i << shiftv[i])
                    lidx = (addr << 10) + jnp.broadcast_to(
                        lbase + j * _L, (_L,)) + lane
                    return a + plsc.load_gather(luts_v, [lidx])

                s = jnp.sum(acc)
                cur = plsc.load_gather(out_v, [rvec])
                plsc.store_scatter(out_v, [rvec], cur + s,
                                   mask=(lane == 0))
            if h == _NH - 1:
                pltpu.sync_copy(
                    out_v, out_hbm.at[pl.ds(row_base + g * _G, _G)])

    return k


_kernel_call = _make_kernel()


@jax.jit
def kernel(x, luts):
    luts_t = luts.reshape(NUM_LUTS, ENTRY_PER_LUT).T.reshape(-1)
    out = _kernel_call(x, luts_t)
    return out.reshape(BATCH, 1)
